# Initial kernel scaffold; baseline (speedup 1.0000x reference)
import jax
import jax.numpy as jnp
from jax.experimental import pallas as pl


def kernel(input_ids, attention_mask, table):
    raise NotImplementedError("write your pallas kernel here")


# speedup vs baseline: 4.1393x; 4.1393x over previous
import functools

import jax
import jax.numpy as jnp
from jax import lax
from jax.experimental import pallas as pl
from jax.experimental.pallas import tpu as pltpu
from jax.experimental.pallas import tpu_sc as plsc

N_CORES = 2
N_SUBCORES = 16
NW = N_CORES * N_SUBCORES

CHUNK = 512


@functools.partial(jax.jit, static_argnums=(2, 3))
def _gather_rows(idx_flat, table, B, D):
    b_per_w = B // NW
    n_chunks = b_per_w // CHUNK
    mesh = plsc.VectorSubcoreMesh(core_axis_name="c", subcore_axis_name="s")

    @functools.partial(
        pl.kernel,
        mesh=mesh,
        out_type=jax.ShapeDtypeStruct((B, D), jnp.float32),
        scratch_types=[
            pltpu.VMEM((CHUNK,), jnp.int32),
            pltpu.VMEM((CHUNK, D), jnp.float32),
            pltpu.SemaphoreType.DMA,
        ],
        compiler_params=pltpu.CompilerParams(use_tc_tiling_on_sc=False),
    )
    def k(idx_hbm, table_hbm, out_hbm, idx_v, rows_v, gsem):
        wid = lax.axis_index("s") * N_CORES + lax.axis_index("c")
        base = wid * b_per_w

        def body(c, carry):
            off = base + c * CHUNK
            pltpu.sync_copy(idx_hbm.at[pl.ds(off, CHUNK)], idx_v)
            pltpu.async_copy(table_hbm.at[idx_v], rows_v, gsem).wait()
            pltpu.sync_copy(rows_v, out_hbm.at[pl.ds(off, CHUNK)])
            return carry

        lax.fori_loop(0, n_chunks, body, 0)

    return k(idx_flat, table)


def kernel(input_ids, attention_mask, table):
    B = input_ids.shape[0] * input_ids.shape[1]
    D = table.shape[1]
    idx_flat = input_ids.reshape(B).astype(jnp.int32)
    rows = _gather_rows(idx_flat, table, B, D)
    token_embeddings = rows.reshape(input_ids.shape[0], input_ids.shape[1], D)
    return (input_ids, token_embeddings, attention_mask)


# speedup vs baseline: 4.4988x; 1.0869x over previous
import functools

import jax
import jax.numpy as jnp
from jax import lax
from jax.experimental import pallas as pl
from jax.experimental.pallas import tpu as pltpu
from jax.experimental.pallas import tpu_sc as plsc

N_CORES = 2
N_SUBCORES = 16
NW = N_CORES * N_SUBCORES

CHUNK = 512


@functools.partial(jax.jit, static_argnums=(2, 3))
def _gather_rows(idx_flat, table, B, D):
    b_per_w = B // NW
    n_chunks = b_per_w // CHUNK
    assert n_chunks % 2 == 0 and n_chunks >= 4
    mesh = plsc.VectorSubcoreMesh(core_axis_name="c", subcore_axis_name="s")

    @functools.partial(
        pl.kernel,
        mesh=mesh,
        out_type=jax.ShapeDtypeStruct((B, D), jnp.float32),
        scratch_types=[
            pltpu.VMEM((b_per_w,), jnp.int32),
            pltpu.VMEM((CHUNK, D), jnp.float32),
            pltpu.VMEM((CHUNK, D), jnp.float32),
            pltpu.SemaphoreType.DMA,
            pltpu.SemaphoreType.DMA,
            pltpu.SemaphoreType.DMA,
            pltpu.SemaphoreType.DMA,
        ],
        compiler_params=pltpu.CompilerParams(use_tc_tiling_on_sc=False),
    )
    def k(idx_hbm, table_hbm, out_hbm, idx_v, rows0, rows1, g0, g1, s0, s1):
        wid = lax.axis_index("s") * N_CORES + lax.axis_index("c")
        base = wid * b_per_w
        rows = (rows0, rows1)
        gsem = (g0, g1)
        ssem = (s0, s1)

        pltpu.sync_copy(idx_hbm.at[pl.ds(base, b_per_w)], idx_v)

        def fire_gather(c, slot):
            pltpu.async_copy(
                table_hbm.at[idx_v.at[pl.ds(c * CHUNK, CHUNK)]], rows[slot], gsem[slot]
            )

        def wait_gather(slot):
            pltpu.make_async_copy(
                table_hbm.at[pl.ds(0, CHUNK)], rows[slot], gsem[slot]
            ).wait()

        def fire_scatter(c, slot):
            pltpu.async_copy(
                rows[slot], out_hbm.at[pl.ds(base + c * CHUNK, CHUNK)], ssem[slot]
            )

        def wait_scatter(slot):
            pltpu.make_async_copy(
                rows[slot], out_hbm.at[pl.ds(base, CHUNK)], ssem[slot]
            ).wait()

        def pair_body(j, first):
            c0 = j * 2
            c1 = c0 + 1
            if not first:
                wait_scatter(0)
            fire_gather(c0, 0)
            if not first:
                wait_gather(1)
                fire_scatter(c0 - 1, 1)
                wait_scatter(1)
            fire_gather(c1, 1)
            wait_gather(0)
            fire_scatter(c0, 0)

        pair_body(0, True)
        lax.fori_loop(1, n_chunks // 2, lambda j, carry: (pair_body(j, False), carry)[1], 0)

        wait_gather(1)
        fire_scatter(n_chunks - 1, 1)
        wait_scatter(0)
        wait_scatter(1)

    return k(idx_flat, table)


def kernel(input_ids, attention_mask, table):
    B = input_ids.shape[0] * input_ids.shape[1]
    D = table.shape[1]
    idx_flat = input_ids.reshape(B).astype(jnp.int32)
    rows = _gather_rows(idx_flat, table, B, D)
    token_embeddings = rows.reshape(input_ids.shape[0], input_ids.shape[1], D)
    return (input_ids, token_embeddings, attention_mask)


# speedup vs baseline: 4.5021x; 1.0007x over previous
import functools

import jax
import jax.numpy as jnp
from jax import lax
from jax.experimental import pallas as pl
from jax.experimental.pallas import tpu as pltpu
from jax.experimental.pallas import tpu_sc as plsc

N_CORES = 2
N_SUBCORES = 16
NW = N_CORES * N_SUBCORES

CHUNK = 256
NBUF = 4
LAG = 2


@functools.partial(jax.jit, static_argnums=(2, 3))
def _gather_rows(idx_flat, table, B, D):
    b_per_w = B // NW
    n_chunks = b_per_w // CHUNK
    assert n_chunks % NBUF == 0 and n_chunks >= 2 * NBUF
    mesh = plsc.VectorSubcoreMesh(core_axis_name="c", subcore_axis_name="s")

    @functools.partial(
        pl.kernel,
        mesh=mesh,
        out_type=jax.ShapeDtypeStruct((B, D), jnp.float32),
        scratch_types=[
            pltpu.VMEM((b_per_w,), jnp.int32),
            pltpu.VMEM((NBUF, CHUNK, D), jnp.float32),
            pltpu.SemaphoreType.DMA((NBUF,)),
            pltpu.SemaphoreType.DMA((NBUF,)),
        ],
        compiler_params=pltpu.CompilerParams(use_tc_tiling_on_sc=False),
    )
    def k(idx_hbm, table_hbm, out_hbm, idx_v, rows_v, gsem, ssem):
        wid = lax.axis_index("s") * N_CORES + lax.axis_index("c")
        base = wid * b_per_w

        pltpu.sync_copy(idx_hbm.at[pl.ds(base, b_per_w)], idx_v)

        def fire_gather(c, b):
            pltpu.async_copy(
                table_hbm.at[idx_v.at[pl.ds(c * CHUNK, CHUNK)]],
                rows_v.at[b],
                gsem.at[b],
            )

        def wait_gather(b):
            pltpu.make_async_copy(
                table_hbm.at[pl.ds(0, CHUNK)], rows_v.at[b], gsem.at[b]
            ).wait()

        def fire_scatter(c, b):
            pltpu.async_copy(
                rows_v.at[b], out_hbm.at[pl.ds(base + c * CHUNK, CHUNK)], ssem.at[b]
            )

        def wait_scatter(b):
            pltpu.make_async_copy(
                rows_v.at[b], out_hbm.at[pl.ds(base, CHUNK)], ssem.at[b]
            ).wait()

        def block(j, first):
            for b in range(NBUF):
                c = j * NBUF + b
                if not first:
                    wait_scatter(b)
                fire_gather(c, b)
                bl = (b - LAG) % NBUF
                if not first or b >= LAG:
                    wait_gather(bl)
                    fire_scatter(c - LAG, bl)

        block(0, True)
        lax.fori_loop(1, n_chunks // NBUF, lambda j, u: (block(j, False), u)[1], 0)

        for c in range(n_chunks - LAG, n_chunks):
            b = c % NBUF
            wait_gather(b)
            fire_scatter(c, b)
        for c in range(n_chunks - NBUF, n_chunks):
            wait_scatter(c % NBUF)

    return k(idx_flat, table)


def kernel(input_ids, attention_mask, table):
    B = input_ids.shape[0] * input_ids.shape[1]
    D = table.shape[1]
    idx_flat = input_ids.reshape(B).astype(jnp.int32)
    rows = _gather_rows(idx_flat, table, B, D)
    token_embeddings = rows.reshape(input_ids.shape[0], input_ids.shape[1], D)
    return (input_ids, token_embeddings, attention_mask)
